# baseline (speedup 1.0000x reference)
import functools

import jax
import jax.numpy as jnp
from jax import lax
from jax.experimental import pallas as pl
from jax.experimental.pallas import tpu as pltpu
from jax.experimental.pallas import tpu_sc as plsc

EMBED = 64
LANES = 16

_info = plsc.get_sparse_core_info()
_NC, _NS = _info.num_cores, _info.num_subcores
_NW = _NC * _NS

CHUNK = 128


def _embedding_gather(Wp, xt, H, B):
    J = xt.shape[0]
    n_chunks = J // CHUNK
    per_w = n_chunks // _NW
    assert per_w * _NW == n_chunks and per_w % 2 == 0
    j_per_w = per_w * CHUNK
    bt_per_h = B // CHUNK

    mesh = plsc.VectorSubcoreMesh(core_axis_name="c", subcore_axis_name="s")

    @functools.partial(
        pl.kernel,
        mesh=mesh,
        out_type=jax.ShapeDtypeStruct((H, EMBED, B), jnp.float32),
        scratch_types=[
            pltpu.VMEM((j_per_w,), jnp.int32),
            pltpu.VMEM((2, CHUNK), jnp.int32),
            pltpu.VMEM((2, CHUNK, 128), jnp.float32),
            pltpu.VMEM((2, EMBED, CHUNK), jnp.float32),
            [pltpu.SemaphoreType.DMA] * 2,
            [pltpu.SemaphoreType.DMA] * 2,
        ],
        compiler_params=pltpu.CompilerParams(
            use_tc_tiling_on_sc=True, needs_layout_passes=False),
    )
    def k(wp_hbm, xt_hbm, out_hbm, idx_v, kbuf, gbuf, obuf, gsems, osems):
        wid = lax.axis_index("s") * _NC + lax.axis_index("c")
        g0 = wid * per_w

        pltpu.sync_copy(xt_hbm.at[pl.ds(wid * j_per_w, j_per_w)], idx_v)

        iota = lax.iota(jnp.int32, LANES)

        def fill_kbuf(t, bi):
            for q in range(CHUNK // LANES):
                xv = idx_v[pl.ds(t * CHUNK + q * LANES, LANES)]
                kbuf[bi, pl.ds(q * LANES, LANES)] = lax.shift_right_logical(
                    xv, 1)

        def start_gather(bi):
            pltpu.async_copy(wp_hbm.at[kbuf.at[bi]], gbuf.at[bi], gsems[bi])

        def wait_gather(bi):
            pltpu.make_async_copy(
                wp_hbm.at[kbuf.at[bi]], gbuf.at[bi], gsems[bi]).wait()

        def out_ref(t):
            g = g0 + t
            h = g // bt_per_h
            bt = g % bt_per_h
            return out_hbm.at[h, pl.ds(0, EMBED), pl.ds(bt * CHUNK, CHUNK)]

        def start_store(t, bi):
            pltpu.async_copy(obuf.at[bi], out_ref(t), osems[bi])

        def wait_store(t, bi):
            pltpu.make_async_copy(obuf.at[bi], out_ref(t), osems[bi]).wait()

        def transpose_chunk(t, bi):
            g2 = gbuf.at[bi]
            rows = []
            cols = []
            for q in range(CHUNK // LANES):
                xv = idx_v[pl.ds(t * CHUNK + q * LANES, LANES)]
                cols.append(
                    lax.shift_left(lax.bitwise_and(xv, 1), 6))
                rows.append(iota + (q * LANES))

            def cbody(c, carry):
                cs = carry
                for q in range(CHUNK // LANES):
                    vals = plsc.load_gather(g2, [rows[q], cs[q]])
                    obuf[bi, c, pl.ds(q * LANES, LANES)] = vals
                return tuple(cv + 1 for cv in cs)

            lax.fori_loop(0, EMBED, cbody, tuple(cols))

        fill_kbuf(0, 0)
        start_gather(0)

        def outer(p, carry):
            for bi in range(2):
                t = p * 2 + bi
                nxt = 1 - bi

                @pl.when(t + 1 < per_w)
                def _():
                    fill_kbuf(t + 1, nxt)
                    start_gather(nxt)

                @pl.when(t >= 2)
                def _():
                    wait_store(t - 2, bi)

                wait_gather(bi)
                transpose_chunk(t, bi)
                start_store(t, bi)
            return carry

        lax.fori_loop(0, per_w // 2, outer, 0)
        wait_store(per_w - 2, 0)
        wait_store(per_w - 1, 1)

    return k(Wp, xt)


def kernel(x, W_embed):
    H_, B_ = x.shape[1], x.shape[0]
    Wp = W_embed.reshape(500000, 128)
    xt = x.T.reshape(-1).astype(jnp.int32)
    out_t = _embedding_gather(Wp, xt, H_, B_)
    return out_t.transpose(2, 0, 1)
